# baseline (speedup 1.0000x reference)
import functools

import jax
import jax.numpy as jnp
from jax import lax
from jax.experimental import pallas as pl
from jax.experimental.pallas import tpu as pltpu
from jax.experimental.pallas import tpu_sc as plsc

N_NODES = 10000
NPAD = 10240
EP = 20480
NB = 10
BM = 200


def _sc_segsum(xmat, srcm, dstm, zeros128):
    n_rows, T = xmat.shape
    nch = T // 128 // 2
    stripe = NPAD // 16
    nz = stripe // 128
    mesh = plsc.VectorSubcoreMesh(core_axis_name="c", subcore_axis_name="s")

    @functools.partial(
        pl.kernel,
        out_type=jax.ShapeDtypeStruct((NPAD, T), jnp.float32),
        mesh=mesh,
        scratch_types=[
            pltpu.VMEM((NB, 128), jnp.int32),
            pltpu.VMEM((NB, 128), jnp.int32),
            pltpu.VMEM((128, 128), jnp.float32),
            pltpu.VMEM((128, 128), jnp.float32),
            pltpu.VMEM_SHARED((NPAD, 128), jnp.float32),
            pltpu.SemaphoreType.DMA,
            pltpu.SemaphoreType.DMA,
        ],
    )
    def k(x_hbm, srcm_hbm, dstm_hbm, z_hbm, out_hbm,
          src_v, dst_v, rows0, rows1, accum, sem0, sem1):
        cid = lax.axis_index("c")
        sid = lax.axis_index("s")
        pltpu.sync_copy(srcm_hbm.at[sid], src_v)
        pltpu.sync_copy(dstm_hbm.at[sid], dst_v)
        row0 = sid * stripe
        bufs = (rows0, rows1)
        sems = (sem0, sem1)

        def chunk_body(ci, carry):
            c0 = (cid * nch + ci) * 128
            zcs = [pltpu.async_copy(
                       z_hbm, accum.at[pl.ds(row0 + z * 128, 128)], sem0)
                   for z in range(nz)]
            for zc in zcs:
                zc.wait()
            plsc.subcore_barrier()

            copies = [
                pltpu.async_copy(
                    x_hbm.at[src_v.at[b], pl.ds(c0, 128)], bufs[b % 2],
                    sems[b % 2])
                for b in [0]
            ]
            for b in range(NB):
                if b + 1 < NB:
                    copies.append(pltpu.async_copy(
                        x_hbm.at[src_v.at[b + 1], pl.ds(c0, 128)],
                        bufs[(b + 1) % 2], sems[(b + 1) % 2]))
                copies[b].wait()
                pltpu.sync_copy(bufs[b % 2], accum.at[dst_v.at[b]], add=True)
            plsc.subcore_barrier()
            fcs = []
            for z in range(nz):
                r = row0 + z * 128
                fcs.append(pltpu.async_copy(
                    accum.at[pl.ds(r, 128)],
                    out_hbm.at[pl.ds(r, 128), pl.ds(c0, 128)], sem1))
            for fc in fcs:
                fc.wait()
            return carry

        lax.fori_loop(0, nch, chunk_body, 0)

    return k(xmat, srcm, dstm, zeros128)


def _mm(x, x2, W, b, act, pre_bias=None, pre_act=None):
    M, K = x.shape
    Nout = W.shape[1]
    b2 = b.reshape(1, Nout)
    two = x2 is not None
    BK = min(K, 2048)
    nk = K // BK

    def body(*refs):
        refs = list(refs)
        xr = refs.pop(0)
        x2r = refs.pop(0) if two else None
        pbr = refs.pop(0) if pre_bias is not None else None
        wr, br, outr = refs
        xx = xr[...]
        if two:
            xx = xx + x2r[...]
        if pre_bias is not None:
            xx = xx + pbr[...]
        if pre_act == "relu":
            xx = jnp.maximum(xx, 0.0)
        k = pl.program_id(1)
        acc = jnp.dot(xx, wr[...], preferred_element_type=jnp.float32,
                      precision=lax.Precision.DEFAULT)

        @pl.when(k == 0)
        def _():
            outr[...] = acc + br[...]

        @pl.when(k > 0)
        def _():
            outr[...] = outr[...] + acc

        @pl.when(k == nk - 1)
        def _():
            if act == "relu":
                outr[...] = jnp.maximum(outr[...], 0.0)
            elif act == "leaky":
                o = outr[...]
                outr[...] = jnp.where(o > 0.0, o, 0.01 * o)

    in_specs = [pl.BlockSpec((BM, BK), lambda i, k: (i, k))]
    args = [x]
    if two:
        in_specs.append(pl.BlockSpec((BM, BK), lambda i, k: (i, k)))
        args.append(x2)
    if pre_bias is not None:
        in_specs.append(pl.BlockSpec((1, BK), lambda i, k: (0, k)))
        args.append(pre_bias.reshape(1, K))
    in_specs += [pl.BlockSpec((BK, Nout), lambda i, k: (k, 0)),
                 pl.BlockSpec((1, Nout), lambda i, k: (0, 0))]
    args += [W, b2]

    return pl.pallas_call(
        body,
        grid=(M // BM, nk),
        in_specs=in_specs,
        out_specs=pl.BlockSpec((BM, Nout), lambda i, k: (i, 0)),
        out_shape=jax.ShapeDtypeStruct((M, Nout), jnp.float32),
    )(*args)


def kernel(x, edge_index, W1a, b1a, W1b, b1b, W2a, b2a, W2b, b2b,
           Wl, bl, Wd1, bd1, Wd2, bd2):
    src = edge_index[0].astype(jnp.int32)
    dst = edge_index[1].astype(jnp.int32)
    pad = EP - src.shape[0]
    srcm = jnp.concatenate(
        [src, jnp.zeros((pad,), jnp.int32)]).reshape(16, NB, 128)
    dstm = jnp.concatenate(
        [dst, jnp.full((pad,), N_NODES, jnp.int32)]).reshape(16, NB, 128)
    zeros128 = jnp.zeros((128, 128), jnp.float32)

    p = _mm(x, None, W1a, jnp.zeros_like(b1a), None)
    aggp = _sc_segsum(p, srcm, dstm, zeros128)
    h = _mm(p, aggp, W1b, b1b, "relu", pre_bias=b1a, pre_act="relu")
    q = _mm(h, None, W2a, jnp.zeros_like(b2a), None)
    aggq = _sc_segsum(q, srcm, dstm, zeros128)
    g = _mm(q, aggq, W2b, b2b, None, pre_bias=b2a, pre_act="relu")
    enc = _mm(g, None, Wl, bl, None)
    d = _mm(enc, None, Wd1, bd1, "leaky")
    dec = _mm(d, None, Wd2, bd2, None)
    return (dec, enc)


# speedup vs baseline: 1.0917x; 1.0917x over previous
import functools

import jax
import jax.numpy as jnp
from jax import lax
from jax.experimental import pallas as pl
from jax.experimental.pallas import tpu as pltpu
from jax.experimental.pallas import tpu_sc as plsc

N_NODES = 10000
NPAD = 10240
EP = 20480
NB = 10
BM = 200


def _sc_segsum(xmat, srcm, dstm, zeros128):
    n_rows, T = xmat.shape
    nch = T // 128 // 2
    stripe = NPAD // 16
    nz = stripe // 128
    mesh = plsc.VectorSubcoreMesh(core_axis_name="c", subcore_axis_name="s")

    @functools.partial(
        pl.kernel,
        out_type=jax.ShapeDtypeStruct((NPAD, T), jnp.float32),
        mesh=mesh,
        scratch_types=[
            pltpu.VMEM((NB, 128), jnp.int32),
            pltpu.VMEM((NB, 128), jnp.int32),
            pltpu.VMEM((128, 128), jnp.float32),
            pltpu.VMEM((128, 128), jnp.float32),
            pltpu.VMEM_SHARED((NPAD, 128), jnp.float32),
            pltpu.SemaphoreType.DMA,
            pltpu.SemaphoreType.DMA,
        ],
    )
    def k(x_hbm, srcm_hbm, dstm_hbm, z_hbm, out_hbm,
          src_v, dst_v, rows0, rows1, accum, sem0, sem1):
        cid = lax.axis_index("c")
        sid = lax.axis_index("s")
        pltpu.sync_copy(srcm_hbm.at[sid], src_v)
        pltpu.sync_copy(dstm_hbm.at[sid], dst_v)
        row0 = sid * stripe
        bufs = (rows0, rows1)
        sems = (sem0, sem1)

        def chunk_body(ci, carry):
            c0 = (cid * nch + ci) * 128
            zcs = [pltpu.async_copy(
                       z_hbm, accum.at[pl.ds(row0 + z * 128, 128)], sem0)
                   for z in range(nz)]
            for zc in zcs:
                zc.wait()
            plsc.subcore_barrier()

            copies = [
                pltpu.async_copy(
                    x_hbm.at[src_v.at[b], pl.ds(c0, 128)], bufs[b % 2],
                    sems[b % 2])
                for b in [0]
            ]
            for b in range(NB):
                if b + 1 < NB:
                    copies.append(pltpu.async_copy(
                        x_hbm.at[src_v.at[b + 1], pl.ds(c0, 128)],
                        bufs[(b + 1) % 2], sems[(b + 1) % 2]))
                copies[b].wait()
                pltpu.sync_copy(bufs[b % 2], accum.at[dst_v.at[b]], add=True)
            plsc.subcore_barrier()
            fcs = []
            for z in range(nz):
                r = row0 + z * 128
                fcs.append(pltpu.async_copy(
                    accum.at[pl.ds(r, 128)],
                    out_hbm.at[pl.ds(r, 128), pl.ds(c0, 128)], sem1))
            for fc in fcs:
                fc.wait()
            return carry

        lax.fori_loop(0, nch, chunk_body, 0)

    return k(xmat, srcm, dstm, zeros128)


def _mm(x, x2, W, b, act, pre_bias=None, pre_act=None):
    M, K = x.shape
    Nout = W.shape[1]
    b2 = b.reshape(1, Nout)
    two = x2 is not None
    BK = min(K, 2048)
    nk = K // BK

    def body(*refs):
        refs = list(refs)
        xr = refs.pop(0)
        x2r = refs.pop(0) if two else None
        pbr = refs.pop(0) if pre_bias is not None else None
        wr, br, outr = refs
        xx = xr[...]
        if two:
            xx = xx + x2r[...]
        if pre_bias is not None:
            xx = xx + pbr[...]
        if pre_act == "relu":
            xx = jnp.maximum(xx, 0.0)
        k = pl.program_id(1)
        acc = jnp.dot(xx, wr[...], preferred_element_type=jnp.float32,
                      precision=lax.Precision.DEFAULT)

        @pl.when(k == 0)
        def _():
            outr[...] = acc + br[...]

        @pl.when(k > 0)
        def _():
            outr[...] = outr[...] + acc

        @pl.when(k == nk - 1)
        def _():
            if act == "relu":
                outr[...] = jnp.maximum(outr[...], 0.0)
            elif act == "leaky":
                o = outr[...]
                outr[...] = jnp.where(o > 0.0, o, 0.01 * o)

    in_specs = [pl.BlockSpec((BM, BK), lambda i, k: (i, k))]
    args = [x]
    if two:
        in_specs.append(pl.BlockSpec((BM, BK), lambda i, k: (i, k)))
        args.append(x2)
    if pre_bias is not None:
        in_specs.append(pl.BlockSpec((1, BK), lambda i, k: (0, k)))
        args.append(pre_bias.reshape(1, K))
    in_specs += [pl.BlockSpec((BK, Nout), lambda i, k: (k, 0)),
                 pl.BlockSpec((1, Nout), lambda i, k: (0, 0))]
    args += [W, b2]

    return pl.pallas_call(
        body,
        grid=(M // BM, nk),
        in_specs=in_specs,
        out_specs=pl.BlockSpec((BM, Nout), lambda i, k: (i, 0)),
        out_shape=jax.ShapeDtypeStruct((M, Nout), jnp.float32),
    )(*args)


def _fused_mid(p, aggp, b1a, W1b, b1b, W2a):
    M, K = p.shape
    N2 = W2a.shape[1]

    def body(pr, ar, pb, wr1, br1, wr2, outr):
        pp = jnp.maximum(pr[...] + ar[...] + pb[...], 0.0)
        h = jnp.dot(pp, wr1[...], preferred_element_type=jnp.float32)
        h = jnp.maximum(h + br1[...], 0.0)
        outr[...] = jnp.dot(h, wr2[...], preferred_element_type=jnp.float32)

    return pl.pallas_call(
        body,
        grid=(M // BM,),
        in_specs=[
            pl.BlockSpec((BM, K), lambda i: (i, 0)),
            pl.BlockSpec((BM, K), lambda i: (i, 0)),
            pl.BlockSpec((1, K), lambda i: (0, 0)),
            pl.BlockSpec((K, K), lambda i: (0, 0)),
            pl.BlockSpec((1, K), lambda i: (0, 0)),
            pl.BlockSpec((K, N2), lambda i: (0, 0)),
        ],
        out_specs=pl.BlockSpec((BM, N2), lambda i: (i, 0)),
        out_shape=jax.ShapeDtypeStruct((M, N2), jnp.float32),
    )(p, aggp, b1a.reshape(1, K), W1b, b1b.reshape(1, K), W2a)


def _fused_tail(q, aggq, b2a, W2b, b2b, Wl, bl, Wd1, bd1, Wd2, bd2):
    M, K = q.shape
    LAT = Wl.shape[1]
    T = Wd2.shape[1]

    def body(qr, ar, pb, w2b, b2, wl, blr, wd1, bd1r, wd2, bd2r,
             dec_ref, enc_ref):
        qq = jnp.maximum(qr[...] + ar[...] + pb[...], 0.0)
        g = jnp.dot(qq, w2b[...], preferred_element_type=jnp.float32) + b2[...]
        enc = jnp.dot(g, wl[...], preferred_element_type=jnp.float32) + blr[...]
        enc_ref[...] = enc
        d = jnp.dot(enc, wd1[...], preferred_element_type=jnp.float32) + bd1r[...]
        d = jnp.where(d > 0.0, d, 0.01 * d)
        dec_ref[...] = jnp.dot(
            d, wd2[...], preferred_element_type=jnp.float32) + bd2r[...]

    return pl.pallas_call(
        body,
        grid=(M // BM,),
        in_specs=[
            pl.BlockSpec((BM, K), lambda i: (i, 0)),
            pl.BlockSpec((BM, K), lambda i: (i, 0)),
            pl.BlockSpec((1, K), lambda i: (0, 0)),
            pl.BlockSpec((K, K), lambda i: (0, 0)),
            pl.BlockSpec((1, K), lambda i: (0, 0)),
            pl.BlockSpec((K, LAT), lambda i: (0, 0)),
            pl.BlockSpec((1, LAT), lambda i: (0, 0)),
            pl.BlockSpec((LAT, K), lambda i: (0, 0)),
            pl.BlockSpec((1, K), lambda i: (0, 0)),
            pl.BlockSpec((K, T), lambda i: (0, 0)),
            pl.BlockSpec((1, T), lambda i: (0, 0)),
        ],
        out_specs=[pl.BlockSpec((BM, T), lambda i: (i, 0)),
                   pl.BlockSpec((BM, LAT), lambda i: (i, 0))],
        out_shape=[jax.ShapeDtypeStruct((M, T), jnp.float32),
                   jax.ShapeDtypeStruct((M, LAT), jnp.float32)],
    )(q, aggq, b2a.reshape(1, K), W2b, b2b.reshape(1, K), Wl,
      bl.reshape(1, LAT), Wd1, bd1.reshape(1, K), Wd2, bd2.reshape(1, T))


def kernel(x, edge_index, W1a, b1a, W1b, b1b, W2a, b2a, W2b, b2b,
           Wl, bl, Wd1, bd1, Wd2, bd2):
    src = edge_index[0].astype(jnp.int32)
    dst = edge_index[1].astype(jnp.int32)
    pad = EP - src.shape[0]
    srcm = jnp.concatenate(
        [src, jnp.zeros((pad,), jnp.int32)]).reshape(16, NB, 128)
    dstm = jnp.concatenate(
        [dst, jnp.full((pad,), N_NODES, jnp.int32)]).reshape(16, NB, 128)
    zeros128 = jnp.zeros((128, 128), jnp.float32)

    p = _mm(x, None, W1a, jnp.zeros_like(b1a), None)
    aggp = _sc_segsum(p, srcm, dstm, zeros128)
    q = _fused_mid(p, aggp, b1a, W1b, b1b, W2a)
    aggq = _sc_segsum(q, srcm, dstm, zeros128)
    dec, enc = _fused_tail(q, aggq, b2a, W2b, b2b, Wl, bl, Wd1, bd1,
                           Wd2, bd2)
    return (dec, enc)


# speedup vs baseline: 1.2388x; 1.1347x over previous
import functools

import jax
import jax.numpy as jnp
from jax import lax
from jax.experimental import pallas as pl
from jax.experimental.pallas import tpu as pltpu
from jax.experimental.pallas import tpu_sc as plsc

N_NODES = 10000
NPAD = 10240
EP = 20480
NB = 10
BM = 200


def _sc_segsum(xmat, srcm, dstm, zeros128):
    n_rows, T = xmat.shape
    nch = T // 128 // 2
    stripe = NPAD // 16
    nz = stripe // 128
    mesh = plsc.VectorSubcoreMesh(core_axis_name="c", subcore_axis_name="s")

    @functools.partial(
        pl.kernel,
        out_type=jax.ShapeDtypeStruct((NPAD, T), jnp.float32),
        mesh=mesh,
        scratch_types=[
            pltpu.VMEM((NB, 128), jnp.int32),
            pltpu.VMEM((NB, 128), jnp.int32),
            pltpu.VMEM((128, 128), jnp.float32),
            pltpu.VMEM((128, 128), jnp.float32),
            pltpu.VMEM_SHARED((NPAD, 128), jnp.float32),
            pltpu.SemaphoreType.DMA,
            pltpu.SemaphoreType.DMA,
        ],
    )
    def k(x_hbm, srcm_hbm, dstm_hbm, z_hbm, out_hbm,
          src_v, dst_v, rows0, rows1, accum, sem0, sem1):
        cid = lax.axis_index("c")
        sid = lax.axis_index("s")
        pltpu.sync_copy(srcm_hbm.at[sid], src_v)
        pltpu.sync_copy(dstm_hbm.at[sid], dst_v)
        row0 = sid * stripe
        bufs = (rows0, rows1)
        sems = (sem0, sem1)

        def chunk_body(ci, carry):
            c0 = (cid * nch + ci) * 128
            zcs = [pltpu.async_copy(
                       z_hbm, accum.at[pl.ds(row0 + z * 128, 128)], sem0)
                   for z in range(nz)]
            for zc in zcs:
                zc.wait()
            plsc.subcore_barrier()

            copies = [
                pltpu.async_copy(
                    x_hbm.at[src_v.at[b], pl.ds(c0, 128)], bufs[b % 2],
                    sems[b % 2])
                for b in [0]
            ]
            for b in range(NB):
                if b + 1 < NB:
                    copies.append(pltpu.async_copy(
                        x_hbm.at[src_v.at[b + 1], pl.ds(c0, 128)],
                        bufs[(b + 1) % 2], sems[(b + 1) % 2]))
                copies[b].wait()
                pltpu.sync_copy(bufs[b % 2], accum.at[dst_v.at[b]], add=True)
            plsc.subcore_barrier()
            fcs = []
            for z in range(nz):
                r = row0 + z * 128
                fcs.append(pltpu.async_copy(
                    accum.at[pl.ds(r, 128)],
                    out_hbm.at[pl.ds(r, 128), pl.ds(c0, 128)], sem1))
            for fc in fcs:
                fc.wait()
            return carry

        lax.fori_loop(0, nch, chunk_body, 0)

    return k(xmat, srcm, dstm, zeros128)


def _mm(x, x2, W, b, act, pre_bias=None, pre_act=None, also_bf16=False):
    M, K = x.shape
    Nout = W.shape[1]
    b2 = b.reshape(1, Nout)
    two = x2 is not None
    BK = min(K, 2048)
    nk = K // BK

    def body(*refs):
        refs = list(refs)
        xr = refs.pop(0)
        x2r = refs.pop(0) if two else None
        pbr = refs.pop(0) if pre_bias is not None else None
        if also_bf16:
            wr, br, outr, bfr = refs
        else:
            wr, br, outr = refs
        xx = xr[...]
        if two:
            xx = xx + x2r[...]
        if pre_bias is not None:
            xx = xx + pbr[...]
        if pre_act == "relu":
            xx = jnp.maximum(xx, 0.0)
        k = pl.program_id(1)
        acc = jnp.dot(xx.astype(jnp.bfloat16), wr[...],
                      preferred_element_type=jnp.float32)

        @pl.when(k == 0)
        def _():
            outr[...] = acc + br[...]

        @pl.when(k > 0)
        def _():
            outr[...] = outr[...] + acc

        @pl.when(k == nk - 1)
        def _():
            if act == "relu":
                outr[...] = jnp.maximum(outr[...], 0.0)
            elif act == "leaky":
                o = outr[...]
                outr[...] = jnp.where(o > 0.0, o, 0.01 * o)
            if also_bf16:
                bfr[...] = outr[...].astype(jnp.bfloat16)

    in_specs = [pl.BlockSpec((BM, BK), lambda i, k: (i, k))]
    args = [x]
    if two:
        in_specs.append(pl.BlockSpec((BM, BK), lambda i, k: (i, k)))
        args.append(x2)
    if pre_bias is not None:
        in_specs.append(pl.BlockSpec((1, BK), lambda i, k: (0, k)))
        args.append(pre_bias.reshape(1, K))
    in_specs += [pl.BlockSpec((BK, Nout), lambda i, k: (k, 0)),
                 pl.BlockSpec((1, Nout), lambda i, k: (0, 0))]
    args += [W, b2]

    out_specs = pl.BlockSpec((BM, Nout), lambda i, k: (i, 0))
    out_shape = jax.ShapeDtypeStruct((M, Nout), jnp.float32)
    if also_bf16:
        out_specs = [out_specs, pl.BlockSpec((BM, Nout), lambda i, k: (i, 0))]
        out_shape = [out_shape, jax.ShapeDtypeStruct((M, Nout), jnp.bfloat16)]
    return pl.pallas_call(
        body,
        grid=(M // BM, nk),
        in_specs=in_specs,
        out_specs=out_specs,
        out_shape=out_shape,
    )(*args)


def _fused_mid(p, aggp, b1a, W1b, b1b, W2a):
    M, K = p.shape
    N2 = W2a.shape[1]

    def body(pr, ar, pb, wr1, br1, wr2, outr):
        pp = jnp.maximum(pr[...] + ar[...] + pb[...], 0.0)
        h = jnp.dot(pp.astype(jnp.bfloat16), wr1[...],
                    preferred_element_type=jnp.float32)
        h = jnp.maximum(h + br1[...], 0.0)
        outr[...] = jnp.dot(h.astype(jnp.bfloat16), wr2[...],
                            preferred_element_type=jnp.float32)

    return pl.pallas_call(
        body,
        grid=(M // BM,),
        in_specs=[
            pl.BlockSpec((BM, K), lambda i: (i, 0)),
            pl.BlockSpec((BM, K), lambda i: (i, 0)),
            pl.BlockSpec((1, K), lambda i: (0, 0)),
            pl.BlockSpec((K, K), lambda i: (0, 0)),
            pl.BlockSpec((1, K), lambda i: (0, 0)),
            pl.BlockSpec((K, N2), lambda i: (0, 0)),
        ],
        out_specs=pl.BlockSpec((BM, N2), lambda i: (i, 0)),
        out_shape=jax.ShapeDtypeStruct((M, N2), jnp.float32),
    )(p, aggp, b1a.reshape(1, K), W1b, b1b.reshape(1, K), W2a)


def _fused_tail(q, aggq, b2a, W2b, b2b, Wl, bl, Wd1, bd1, Wd2, bd2):
    M, K = q.shape
    LAT = Wl.shape[1]
    T = Wd2.shape[1]

    def body(qr, ar, pb, w2b, b2, wl, blr, wd1, bd1r, wd2, bd2r,
             dec_ref, enc_ref):
        qq = jnp.maximum(qr[...] + ar[...] + pb[...], 0.0)
        g = jnp.dot(qq.astype(jnp.bfloat16), w2b[...],
                    preferred_element_type=jnp.float32) + b2[...]
        enc = jnp.dot(g.astype(jnp.bfloat16), wl[...],
                      preferred_element_type=jnp.float32) + blr[...]
        enc_ref[...] = enc
        d = jnp.dot(enc.astype(jnp.bfloat16), wd1[...],
                    preferred_element_type=jnp.float32) + bd1r[...]
        d = jnp.where(d > 0.0, d, 0.01 * d)
        dec_ref[...] = jnp.dot(
            d.astype(jnp.bfloat16), wd2[...],
            preferred_element_type=jnp.float32) + bd2r[...]

    return pl.pallas_call(
        body,
        grid=(M // BM,),
        in_specs=[
            pl.BlockSpec((BM, K), lambda i: (i, 0)),
            pl.BlockSpec((BM, K), lambda i: (i, 0)),
            pl.BlockSpec((1, K), lambda i: (0, 0)),
            pl.BlockSpec((K, K), lambda i: (0, 0)),
            pl.BlockSpec((1, K), lambda i: (0, 0)),
            pl.BlockSpec((K, LAT), lambda i: (0, 0)),
            pl.BlockSpec((1, LAT), lambda i: (0, 0)),
            pl.BlockSpec((LAT, K), lambda i: (0, 0)),
            pl.BlockSpec((1, K), lambda i: (0, 0)),
            pl.BlockSpec((K, T), lambda i: (0, 0)),
            pl.BlockSpec((1, T), lambda i: (0, 0)),
        ],
        out_specs=[pl.BlockSpec((BM, T), lambda i: (i, 0)),
                   pl.BlockSpec((BM, LAT), lambda i: (i, 0))],
        out_shape=[jax.ShapeDtypeStruct((M, T), jnp.float32),
                   jax.ShapeDtypeStruct((M, LAT), jnp.float32)],
    )(q, aggq, b2a.reshape(1, K), W2b, b2b.reshape(1, K), Wl,
      bl.reshape(1, LAT), Wd1, bd1.reshape(1, K), Wd2, bd2.reshape(1, T))


def kernel(x, edge_index, W1a, b1a, W1b, b1b, W2a, b2a, W2b, b2b,
           Wl, bl, Wd1, bd1, Wd2, bd2):
    src = edge_index[0].astype(jnp.int32)
    dst = edge_index[1].astype(jnp.int32)
    pad = EP - src.shape[0]
    srcm = jnp.concatenate(
        [src, jnp.zeros((pad,), jnp.int32)]).reshape(16, NB, 128)
    dstm = jnp.concatenate(
        [dst, jnp.full((pad,), N_NODES, jnp.int32)]).reshape(16, NB, 128)
    zeros128 = jnp.zeros((128, 128), jnp.float32)
    bf = jnp.bfloat16
    W1a_b, W1b_b, W2a_b = W1a.astype(bf), W1b.astype(bf), W2a.astype(bf)
    W2b_b, Wl_b = W2b.astype(bf), Wl.astype(bf)
    Wd1_b, Wd2_b = Wd1.astype(bf), Wd2.astype(bf)

    p = _mm(x, None, W1a_b, jnp.zeros_like(b1a), None)
    aggp = _sc_segsum(p, srcm, dstm, zeros128)
    q = _fused_mid(p, aggp, b1a, W1b_b, b1b, W2a_b)
    aggq = _sc_segsum(q, srcm, dstm, zeros128)
    dec, enc = _fused_tail(q, aggq, b2a, W2b_b, b2b, Wl_b, bl, Wd1_b, bd1,
                           Wd2_b, bd2)
    return (dec, enc)


# speedup vs baseline: 1.2390x; 1.0002x over previous
import functools

import jax
import jax.numpy as jnp
from jax import lax
from jax.experimental import pallas as pl
from jax.experimental.pallas import tpu as pltpu
from jax.experimental.pallas import tpu_sc as plsc

N_NODES = 10000
NPAD = 10240
EP = 20480
NB = 10
BM = 200


def _sc_segsum(xmat, srcm, dstm, zeros128):
    n_rows, T = xmat.shape
    nch = T // 128 // 2
    stripe = NPAD // 16
    nz = stripe // 128
    mesh = plsc.VectorSubcoreMesh(core_axis_name="c", subcore_axis_name="s")

    @functools.partial(
        pl.kernel,
        out_type=jax.ShapeDtypeStruct((NPAD, T), jnp.float32),
        mesh=mesh,
        scratch_types=[
            pltpu.VMEM((NB, 128), jnp.int32),
            pltpu.VMEM((NB, 128), jnp.int32),
            pltpu.VMEM((128, 128), jnp.float32),
            pltpu.VMEM((128, 128), jnp.float32),
            pltpu.VMEM_SHARED((NPAD, 128), jnp.float32),
            pltpu.SemaphoreType.DMA,
            pltpu.SemaphoreType.DMA,
            pltpu.SemaphoreType.DMA,
            pltpu.SemaphoreType.DMA,
        ],
    )
    def k(x_hbm, srcm_hbm, dstm_hbm, z_hbm, out_hbm,
          src_v, dst_v, rows0, rows1, accum, sem0, sem1, asem0, asem1):
        cid = lax.axis_index("c")
        sid = lax.axis_index("s")
        pltpu.sync_copy(srcm_hbm.at[sid], src_v)
        pltpu.sync_copy(dstm_hbm.at[sid], dst_v)
        row0 = sid * stripe
        bufs = (rows0, rows1)
        sems = (sem0, sem1)
        asems = (asem0, asem1)

        def chunk_body(ci, carry):
            c0 = (cid * nch + ci) * 128
            zcs = [pltpu.async_copy(
                       z_hbm, accum.at[pl.ds(row0 + z * 128, 128)], sem0)
                   for z in range(nz)]
            for zc in zcs:
                zc.wait()
            plsc.subcore_barrier()

            copies = [
                pltpu.async_copy(
                    x_hbm.at[src_v.at[b], pl.ds(c0, 128)], bufs[b % 2],
                    sems[b % 2])
                for b in [0]
            ]
            adds = []
            for b in range(NB):
                if b + 1 < NB:
                    if b >= 1:
                        adds[b - 1].wait()
                    copies.append(pltpu.async_copy(
                        x_hbm.at[src_v.at[b + 1], pl.ds(c0, 128)],
                        bufs[(b + 1) % 2], sems[(b + 1) % 2]))
                copies[b].wait()
                adds.append(pltpu.async_copy(
                    bufs[b % 2], accum.at[dst_v.at[b]], asems[b % 2],
                    add=True))
            adds[NB - 2].wait()
            adds[NB - 1].wait()
            plsc.subcore_barrier()
            fcs = []
            for z in range(nz):
                r = row0 + z * 128
                fcs.append(pltpu.async_copy(
                    accum.at[pl.ds(r, 128)],
                    out_hbm.at[pl.ds(r, 128), pl.ds(c0, 128)], sem1))
            for fc in fcs:
                fc.wait()
            return carry

        lax.fori_loop(0, nch, chunk_body, 0)

    return k(xmat, srcm, dstm, zeros128)


def _mm(x, x2, W, b, act, pre_bias=None, pre_act=None, also_bf16=False):
    M, K = x.shape
    Nout = W.shape[1]
    b2 = b.reshape(1, Nout)
    two = x2 is not None
    BK = min(K, 2048)
    nk = K // BK

    def body(*refs):
        refs = list(refs)
        xr = refs.pop(0)
        x2r = refs.pop(0) if two else None
        pbr = refs.pop(0) if pre_bias is not None else None
        if also_bf16:
            wr, br, outr, bfr = refs
        else:
            wr, br, outr = refs
        xx = xr[...]
        if two:
            xx = xx + x2r[...]
        if pre_bias is not None:
            xx = xx + pbr[...]
        if pre_act == "relu":
            xx = jnp.maximum(xx, 0.0)
        k = pl.program_id(1)
        acc = jnp.dot(xx.astype(jnp.bfloat16), wr[...],
                      preferred_element_type=jnp.float32)

        @pl.when(k == 0)
        def _():
            outr[...] = acc + br[...]

        @pl.when(k > 0)
        def _():
            outr[...] = outr[...] + acc

        @pl.when(k == nk - 1)
        def _():
            if act == "relu":
                outr[...] = jnp.maximum(outr[...], 0.0)
            elif act == "leaky":
                o = outr[...]
                outr[...] = jnp.where(o > 0.0, o, 0.01 * o)
            if also_bf16:
                bfr[...] = outr[...].astype(jnp.bfloat16)

    in_specs = [pl.BlockSpec((BM, BK), lambda i, k: (i, k))]
    args = [x]
    if two:
        in_specs.append(pl.BlockSpec((BM, BK), lambda i, k: (i, k)))
        args.append(x2)
    if pre_bias is not None:
        in_specs.append(pl.BlockSpec((1, BK), lambda i, k: (0, k)))
        args.append(pre_bias.reshape(1, K))
    in_specs += [pl.BlockSpec((BK, Nout), lambda i, k: (k, 0)),
                 pl.BlockSpec((1, Nout), lambda i, k: (0, 0))]
    args += [W, b2]

    out_specs = pl.BlockSpec((BM, Nout), lambda i, k: (i, 0))
    out_shape = jax.ShapeDtypeStruct((M, Nout), jnp.float32)
    if also_bf16:
        out_specs = [out_specs, pl.BlockSpec((BM, Nout), lambda i, k: (i, 0))]
        out_shape = [out_shape, jax.ShapeDtypeStruct((M, Nout), jnp.bfloat16)]
    return pl.pallas_call(
        body,
        grid=(M // BM, nk),
        in_specs=in_specs,
        out_specs=out_specs,
        out_shape=out_shape,
    )(*args)


def _fused_mid(p, aggp, b1a, W1b, b1b, W2a):
    M, K = p.shape
    N2 = W2a.shape[1]

    def body(pr, ar, pb, wr1, br1, wr2, outr):
        pp = jnp.maximum(pr[...] + ar[...] + pb[...], 0.0)
        h = jnp.dot(pp.astype(jnp.bfloat16), wr1[...],
                    preferred_element_type=jnp.float32)
        h = jnp.maximum(h + br1[...], 0.0)
        outr[...] = jnp.dot(h.astype(jnp.bfloat16), wr2[...],
                            preferred_element_type=jnp.float32)

    return pl.pallas_call(
        body,
        grid=(M // BM,),
        in_specs=[
            pl.BlockSpec((BM, K), lambda i: (i, 0)),
            pl.BlockSpec((BM, K), lambda i: (i, 0)),
            pl.BlockSpec((1, K), lambda i: (0, 0)),
            pl.BlockSpec((K, K), lambda i: (0, 0)),
            pl.BlockSpec((1, K), lambda i: (0, 0)),
            pl.BlockSpec((K, N2), lambda i: (0, 0)),
        ],
        out_specs=pl.BlockSpec((BM, N2), lambda i: (i, 0)),
        out_shape=jax.ShapeDtypeStruct((M, N2), jnp.float32),
    )(p, aggp, b1a.reshape(1, K), W1b, b1b.reshape(1, K), W2a)


def _fused_tail(q, aggq, b2a, W2b, b2b, Wl, bl, Wd1, bd1, Wd2, bd2):
    M, K = q.shape
    LAT = Wl.shape[1]
    T = Wd2.shape[1]

    def body(qr, ar, pb, w2b, b2, wl, blr, wd1, bd1r, wd2, bd2r,
             dec_ref, enc_ref):
        qq = jnp.maximum(qr[...] + ar[...] + pb[...], 0.0)
        g = jnp.dot(qq.astype(jnp.bfloat16), w2b[...],
                    preferred_element_type=jnp.float32) + b2[...]
        enc = jnp.dot(g.astype(jnp.bfloat16), wl[...],
                      preferred_element_type=jnp.float32) + blr[...]
        enc_ref[...] = enc
        d = jnp.dot(enc.astype(jnp.bfloat16), wd1[...],
                    preferred_element_type=jnp.float32) + bd1r[...]
        d = jnp.where(d > 0.0, d, 0.01 * d)
        dec_ref[...] = jnp.dot(
            d.astype(jnp.bfloat16), wd2[...],
            preferred_element_type=jnp.float32) + bd2r[...]

    return pl.pallas_call(
        body,
        grid=(M // BM,),
        in_specs=[
            pl.BlockSpec((BM, K), lambda i: (i, 0)),
            pl.BlockSpec((BM, K), lambda i: (i, 0)),
            pl.BlockSpec((1, K), lambda i: (0, 0)),
            pl.BlockSpec((K, K), lambda i: (0, 0)),
            pl.BlockSpec((1, K), lambda i: (0, 0)),
            pl.BlockSpec((K, LAT), lambda i: (0, 0)),
            pl.BlockSpec((1, LAT), lambda i: (0, 0)),
            pl.BlockSpec((LAT, K), lambda i: (0, 0)),
            pl.BlockSpec((1, K), lambda i: (0, 0)),
            pl.BlockSpec((K, T), lambda i: (0, 0)),
            pl.BlockSpec((1, T), lambda i: (0, 0)),
        ],
        out_specs=[pl.BlockSpec((BM, T), lambda i: (i, 0)),
                   pl.BlockSpec((BM, LAT), lambda i: (i, 0))],
        out_shape=[jax.ShapeDtypeStruct((M, T), jnp.float32),
                   jax.ShapeDtypeStruct((M, LAT), jnp.float32)],
    )(q, aggq, b2a.reshape(1, K), W2b, b2b.reshape(1, K), Wl,
      bl.reshape(1, LAT), Wd1, bd1.reshape(1, K), Wd2, bd2.reshape(1, T))


def kernel(x, edge_index, W1a, b1a, W1b, b1b, W2a, b2a, W2b, b2b,
           Wl, bl, Wd1, bd1, Wd2, bd2):
    src = edge_index[0].astype(jnp.int32)
    dst = edge_index[1].astype(jnp.int32)
    pad = EP - src.shape[0]
    srcm = jnp.concatenate(
        [src, jnp.zeros((pad,), jnp.int32)]).reshape(16, NB, 128)
    dstm = jnp.concatenate(
        [dst, jnp.full((pad,), N_NODES, jnp.int32)]).reshape(16, NB, 128)
    zeros128 = jnp.zeros((128, 128), jnp.float32)
    bf = jnp.bfloat16
    W1a_b, W1b_b, W2a_b = W1a.astype(bf), W1b.astype(bf), W2a.astype(bf)
    W2b_b, Wl_b = W2b.astype(bf), Wl.astype(bf)
    Wd1_b, Wd2_b = Wd1.astype(bf), Wd2.astype(bf)

    p = _mm(x, None, W1a_b, jnp.zeros_like(b1a), None)
    aggp = _sc_segsum(p, srcm, dstm, zeros128)
    q = _fused_mid(p, aggp, b1a, W1b_b, b1b, W2a_b)
    aggq = _sc_segsum(q, srcm, dstm, zeros128)
    dec, enc = _fused_tail(q, aggq, b2a, W2b_b, b2b, Wl_b, bl, Wd1_b, bd1,
                           Wd2_b, bd2)
    return (dec, enc)


# speedup vs baseline: 1.3994x; 1.1294x over previous
import functools

import jax
import jax.numpy as jnp
from jax import lax
from jax.experimental import pallas as pl
from jax.experimental.pallas import tpu as pltpu
from jax.experimental.pallas import tpu_sc as plsc

N_NODES = 10000
NPAD = 10240
EP = 20480
NB = 10
BM = 400


def _sc_segsum(xmat, srcm, dstm, zeros128):
    n_rows, T = xmat.shape
    nch = T // 128 // 2
    stripe = NPAD // 16
    nz = stripe // 128
    mesh = plsc.VectorSubcoreMesh(core_axis_name="c", subcore_axis_name="s")

    @functools.partial(
        pl.kernel,
        out_type=jax.ShapeDtypeStruct((NPAD, T), jnp.float32),
        mesh=mesh,
        scratch_types=[
            pltpu.VMEM((NB, 128), jnp.int32),
            pltpu.VMEM((NB, 128), jnp.int32),
            pltpu.VMEM((128, 128), jnp.float32),
            pltpu.VMEM((128, 128), jnp.float32),
            pltpu.VMEM_SHARED((NPAD, 128), jnp.float32),
            pltpu.SemaphoreType.DMA,
            pltpu.SemaphoreType.DMA,
            pltpu.SemaphoreType.DMA,
            pltpu.SemaphoreType.DMA,
        ],
    )
    def k(x_hbm, srcm_hbm, dstm_hbm, z_hbm, out_hbm,
          src_v, dst_v, rows0, rows1, accum, sem0, sem1, asem0, asem1):
        cid = lax.axis_index("c")
        sid = lax.axis_index("s")
        pltpu.sync_copy(srcm_hbm.at[sid], src_v)
        pltpu.sync_copy(dstm_hbm.at[sid], dst_v)
        row0 = sid * stripe
        bufs = (rows0, rows1)
        sems = (sem0, sem1)
        asems = (asem0, asem1)

        def chunk_body(ci, carry):
            c0 = (cid * nch + ci) * 128
            zcs = [pltpu.async_copy(
                       z_hbm, accum.at[pl.ds(row0 + z * 128, 128)], sem0)
                   for z in range(nz)]
            for zc in zcs:
                zc.wait()
            plsc.subcore_barrier()

            copies = [
                pltpu.async_copy(
                    x_hbm.at[src_v.at[b], pl.ds(c0, 128)], bufs[b % 2],
                    sems[b % 2])
                for b in [0]
            ]
            adds = []
            for b in range(NB):
                if b + 1 < NB:
                    if b >= 1:
                        adds[b - 1].wait()
                    copies.append(pltpu.async_copy(
                        x_hbm.at[src_v.at[b + 1], pl.ds(c0, 128)],
                        bufs[(b + 1) % 2], sems[(b + 1) % 2]))
                copies[b].wait()
                adds.append(pltpu.async_copy(
                    bufs[b % 2], accum.at[dst_v.at[b]], asems[b % 2],
                    add=True))
            adds[NB - 2].wait()
            adds[NB - 1].wait()
            plsc.subcore_barrier()
            fcs = []
            for z in range(nz):
                r = row0 + z * 128
                fcs.append(pltpu.async_copy(
                    accum.at[pl.ds(r, 128)],
                    out_hbm.at[pl.ds(r, 128), pl.ds(c0, 128)], sem1))
            for fc in fcs:
                fc.wait()
            return carry

        lax.fori_loop(0, nch, chunk_body, 0)

    return k(xmat, srcm, dstm, zeros128)


def _mm(x, x2, W, b, act, pre_bias=None, pre_act=None, also_bf16=False):
    M, K = x.shape
    Nout = W.shape[1]
    b2 = b.reshape(1, Nout)
    two = x2 is not None
    BK = min(K, 4096)
    nk = K // BK

    def body(*refs):
        refs = list(refs)
        xr = refs.pop(0)
        x2r = refs.pop(0) if two else None
        pbr = refs.pop(0) if pre_bias is not None else None
        if also_bf16:
            wr, br, outr, bfr = refs
        else:
            wr, br, outr = refs
        xx = xr[...]
        if two:
            xx = xx + x2r[...]
        if pre_bias is not None:
            xx = xx + pbr[...]
        if pre_act == "relu":
            xx = jnp.maximum(xx, 0.0)
        k = pl.program_id(1)
        acc = jnp.dot(xx.astype(jnp.bfloat16), wr[...],
                      preferred_element_type=jnp.float32)

        @pl.when(k == 0)
        def _():
            outr[...] = acc + br[...]

        @pl.when(k > 0)
        def _():
            outr[...] = outr[...] + acc

        @pl.when(k == nk - 1)
        def _():
            if act == "relu":
                outr[...] = jnp.maximum(outr[...], 0.0)
            elif act == "leaky":
                o = outr[...]
                outr[...] = jnp.where(o > 0.0, o, 0.01 * o)
            if also_bf16:
                bfr[...] = outr[...].astype(jnp.bfloat16)

    in_specs = [pl.BlockSpec((BM, BK), lambda i, k: (i, k))]
    args = [x]
    if two:
        in_specs.append(pl.BlockSpec((BM, BK), lambda i, k: (i, k)))
        args.append(x2)
    if pre_bias is not None:
        in_specs.append(pl.BlockSpec((1, BK), lambda i, k: (0, k)))
        args.append(pre_bias.reshape(1, K))
    in_specs += [pl.BlockSpec((BK, Nout), lambda i, k: (k, 0)),
                 pl.BlockSpec((1, Nout), lambda i, k: (0, 0))]
    args += [W, b2]

    out_specs = pl.BlockSpec((BM, Nout), lambda i, k: (i, 0))
    out_shape = jax.ShapeDtypeStruct((M, Nout), jnp.float32)
    if also_bf16:
        out_specs = [out_specs, pl.BlockSpec((BM, Nout), lambda i, k: (i, 0))]
        out_shape = [out_shape, jax.ShapeDtypeStruct((M, Nout), jnp.bfloat16)]
    return pl.pallas_call(
        body,
        grid=(M // BM, nk),
        in_specs=in_specs,
        out_specs=out_specs,
        out_shape=out_shape,
    )(*args)


def _fused_mid(p, aggp, b1a, W1b, b1b, W2a):
    M, K = p.shape
    N2 = W2a.shape[1]

    def body(pr, ar, pb, wr1, br1, wr2, outr):
        pp = jnp.maximum(pr[...] + ar[...] + pb[...], 0.0)
        h = jnp.dot(pp.astype(jnp.bfloat16), wr1[...],
                    preferred_element_type=jnp.float32)
        h = jnp.maximum(h + br1[...], 0.0)
        outr[...] = jnp.dot(h.astype(jnp.bfloat16), wr2[...],
                            preferred_element_type=jnp.float32)

    return pl.pallas_call(
        body,
        grid=(M // BM,),
        in_specs=[
            pl.BlockSpec((BM, K), lambda i: (i, 0)),
            pl.BlockSpec((BM, K), lambda i: (i, 0)),
            pl.BlockSpec((1, K), lambda i: (0, 0)),
            pl.BlockSpec((K, K), lambda i: (0, 0)),
            pl.BlockSpec((1, K), lambda i: (0, 0)),
            pl.BlockSpec((K, N2), lambda i: (0, 0)),
        ],
        out_specs=pl.BlockSpec((BM, N2), lambda i: (i, 0)),
        out_shape=jax.ShapeDtypeStruct((M, N2), jnp.float32),
    )(p, aggp, b1a.reshape(1, K), W1b, b1b.reshape(1, K), W2a)


def _fused_tail(q, aggq, b2a, W2b, b2b, Wl, bl, Wd1, bd1, Wd2, bd2):
    M, K = q.shape
    LAT = Wl.shape[1]
    T = Wd2.shape[1]

    def body(qr, ar, pb, w2b, b2, wl, blr, wd1, bd1r, wd2, bd2r,
             dec_ref, enc_ref):
        qq = jnp.maximum(qr[...] + ar[...] + pb[...], 0.0)
        g = jnp.dot(qq.astype(jnp.bfloat16), w2b[...],
                    preferred_element_type=jnp.float32) + b2[...]
        enc = jnp.dot(g.astype(jnp.bfloat16), wl[...],
                      preferred_element_type=jnp.float32) + blr[...]
        enc_ref[...] = enc
        d = jnp.dot(enc.astype(jnp.bfloat16), wd1[...],
                    preferred_element_type=jnp.float32) + bd1r[...]
        d = jnp.where(d > 0.0, d, 0.01 * d)
        dec_ref[...] = jnp.dot(
            d.astype(jnp.bfloat16), wd2[...],
            preferred_element_type=jnp.float32) + bd2r[...]

    return pl.pallas_call(
        body,
        grid=(M // BM,),
        in_specs=[
            pl.BlockSpec((BM, K), lambda i: (i, 0)),
            pl.BlockSpec((BM, K), lambda i: (i, 0)),
            pl.BlockSpec((1, K), lambda i: (0, 0)),
            pl.BlockSpec((K, K), lambda i: (0, 0)),
            pl.BlockSpec((1, K), lambda i: (0, 0)),
            pl.BlockSpec((K, LAT), lambda i: (0, 0)),
            pl.BlockSpec((1, LAT), lambda i: (0, 0)),
            pl.BlockSpec((LAT, K), lambda i: (0, 0)),
            pl.BlockSpec((1, K), lambda i: (0, 0)),
            pl.BlockSpec((K, T), lambda i: (0, 0)),
            pl.BlockSpec((1, T), lambda i: (0, 0)),
        ],
        out_specs=[pl.BlockSpec((BM, T), lambda i: (i, 0)),
                   pl.BlockSpec((BM, LAT), lambda i: (i, 0))],
        out_shape=[jax.ShapeDtypeStruct((M, T), jnp.float32),
                   jax.ShapeDtypeStruct((M, LAT), jnp.float32)],
    )(q, aggq, b2a.reshape(1, K), W2b, b2b.reshape(1, K), Wl,
      bl.reshape(1, LAT), Wd1, bd1.reshape(1, K), Wd2, bd2.reshape(1, T))


def kernel(x, edge_index, W1a, b1a, W1b, b1b, W2a, b2a, W2b, b2b,
           Wl, bl, Wd1, bd1, Wd2, bd2):
    src = edge_index[0].astype(jnp.int32)
    dst = edge_index[1].astype(jnp.int32)
    pad = EP - src.shape[0]
    srcm = jnp.concatenate(
        [src, jnp.zeros((pad,), jnp.int32)]).reshape(16, NB, 128)
    dstm = jnp.concatenate(
        [dst, jnp.full((pad,), N_NODES, jnp.int32)]).reshape(16, NB, 128)
    zeros128 = jnp.zeros((128, 128), jnp.float32)
    bf = jnp.bfloat16
    W1a_b, W1b_b, W2a_b = W1a.astype(bf), W1b.astype(bf), W2a.astype(bf)
    W2b_b, Wl_b = W2b.astype(bf), Wl.astype(bf)
    Wd1_b, Wd2_b = Wd1.astype(bf), Wd2.astype(bf)

    p = _mm(x, None, W1a_b, jnp.zeros_like(b1a), None)
    aggp = _sc_segsum(p, srcm, dstm, zeros128)
    q = _fused_mid(p, aggp, b1a, W1b_b, b1b, W2a_b)
    aggq = _sc_segsum(q, srcm, dstm, zeros128)
    dec, enc = _fused_tail(q, aggq, b2a, W2b_b, b2b, Wl_b, bl, Wd1_b, bd1,
                           Wd2_b, bd2)
    return (dec, enc)
